# baseline (speedup 1.0000x reference)
import functools

import jax
import jax.numpy as jnp
from jax import lax
from jax.experimental import pallas as pl
from jax.experimental.pallas import tpu as pltpu
from jax.experimental.pallas import tpu_sc as plsc

D = 128
E = 160000
E2 = 2 * E
L = 16
NC, NS = 2, 16
NW = NC * NS
C = 128
K = 80
PAD_E = (NW * (K + 2) - 1) * C + C

_PERM_DNUMS = lax.GatherDimensionNumbers(
    offset_dims=(), collapsed_slice_dims=(0,), start_index_map=(0,))


def _permute(x, idx):
    return lax.gather(x, idx[:, None], _PERM_DNUMS, slice_sizes=(1,),
                      mode=lax.GatherScatterMode.PROMISE_IN_BOUNDS)


@functools.partial(
    pl.kernel,
    out_type=jax.ShapeDtypeStruct((PAD_E,), jnp.float32),
    mesh=plsc.VectorSubcoreMesh(core_axis_name="c", subcore_axis_name="s"),
    scratch_types=[
        pltpu.VMEM((C,), jnp.int32), pltpu.VMEM((C,), jnp.int32),
        pltpu.VMEM((C,), jnp.int32), pltpu.VMEM((C,), jnp.int32),
        pltpu.VMEM((C, D), jnp.float32), pltpu.VMEM((C, D), jnp.float32),
        pltpu.VMEM((C, D), jnp.float32), pltpu.VMEM((C, D), jnp.float32),
        pltpu.VMEM((C,), jnp.float32), pltpu.VMEM((C,), jnp.float32),
        pltpu.SemaphoreType.DMA, pltpu.SemaphoreType.DMA,
        pltpu.SemaphoreType.DMA, pltpu.SemaphoreType.DMA,
        pltpu.SemaphoreType.DMA, pltpu.SemaphoreType.DMA,
    ],
)
def _edge_scores(user_hbm, item_hbm, ue_hbm, ie_hbm, out_hbm,
                 uidx0, uidx1, iidx0, iidx1,
                 urows0, urows1, vrows0, vrows1,
                 outv0, outv1,
                 semi0, semi1, semg0, semg1, semo0, semo1):
    wid = lax.axis_index("s") * NC + lax.axis_index("c")
    lane = lax.iota(jnp.int32, L)
    perms = [jnp.bitwise_xor(lane, s) for s in (8, 4, 2, 1)]

    uidx = (uidx0, uidx1)
    iidx = (iidx0, iidx1)
    urows = (urows0, urows1)
    vrows = (vrows0, vrows1)
    outv = (outv0, outv1)
    semi = (semi0, semi1)
    semg = (semg0, semg1)
    semo = (semo0, semo1)

    def base_of(k):
        return (wid + k * NW) * C

    def start_idx(k, b):
        pltpu.async_copy(ue_hbm.at[pl.ds(base_of(k), C)], uidx[b], semi[b])
        pltpu.async_copy(ie_hbm.at[pl.ds(base_of(k), C)], iidx[b], semi[b])

    def wait_idx(b):
        pltpu.make_async_copy(ue_hbm.at[pl.ds(0, C)], uidx[b], semi[b]).wait()
        pltpu.make_async_copy(ie_hbm.at[pl.ds(0, C)], iidx[b], semi[b]).wait()

    def start_gather(b):
        pltpu.async_copy(user_hbm.at[uidx[b]], urows[b], semg[b])
        pltpu.async_copy(item_hbm.at[iidx[b]], vrows[b], semg[b])

    def wait_gather(b):
        pltpu.make_async_copy(user_hbm.at[uidx[b]], urows[b], semg[b]).wait()
        pltpu.make_async_copy(item_hbm.at[iidx[b]], vrows[b], semg[b]).wait()

    def compute(b):
        def grp_body(g, _):
            out_vec = jnp.zeros((L,), jnp.float32)
            for e in range(L):
                row = g * L + e
                acc = urows[b][row, pl.ds(0, L)] * vrows[b][row, pl.ds(0, L)]
                for kk in range(1, D // L):
                    a = urows[b][row, pl.ds(kk * L, L)]
                    v = vrows[b][row, pl.ds(kk * L, L)]
                    acc = acc + a * v
                for p in perms:
                    acc = acc + _permute(acc, p)
                out_vec = jnp.where(lane == e, acc, out_vec)
            outv[b][pl.ds(g * L, L)] = out_vec
            return 0

        lax.fori_loop(0, C // L, grp_body, 0)

    def start_out(k, b):
        pltpu.async_copy(outv[b], out_hbm.at[pl.ds(base_of(k), C)], semo[b])

    def wait_out(b):
        pltpu.make_async_copy(outv[b], out_hbm.at[pl.ds(0, C)],
                              semo[b]).wait()

    def loop_body(k, _):
        pltpu.sync_copy(ue_hbm.at[pl.ds(base_of(k), C)], uidx[0])
        pltpu.sync_copy(ie_hbm.at[pl.ds(base_of(k), C)], iidx[0])
        start_gather(0)
        wait_gather(0)
        compute(0)
        pltpu.sync_copy(outv[0], out_hbm.at[pl.ds(base_of(k), C)])
        return 0

    lax.fori_loop(0, K, loop_body, 0)


def kernel(user_embedding, item_embedding, pos_edges, neg_edges):
    pad = jnp.zeros((PAD_E - E2,), jnp.int32)
    ue = jnp.concatenate([pos_edges[0], neg_edges[0], pad])
    ie = jnp.concatenate([pos_edges[1], neg_edges[1], pad])
    out = _edge_scores(user_embedding, item_embedding, ue, ie)
    return (out[:E, None], out[E:E2, None])


# speedup vs baseline: 1.7872x; 1.7872x over previous
import functools

import jax
import jax.numpy as jnp
from jax import lax
from jax.experimental import pallas as pl
from jax.experimental.pallas import tpu as pltpu
from jax.experimental.pallas import tpu_sc as plsc

D = 128
E = 160000
L = 16
NC, NS = 2, 16
NW = NC * NS
C = 128
NCHUNKS = E // C
BASE_PER_W = NCHUNKS // NW
EXTRA = NCHUNKS - BASE_PER_W * NW

_PERM_DNUMS = lax.GatherDimensionNumbers(
    offset_dims=(), collapsed_slice_dims=(0,), start_index_map=(0,))


def _permute(x, idx):
    return lax.gather(x, idx[:, None], _PERM_DNUMS, slice_sizes=(1,),
                      mode=lax.GatherScatterMode.PROMISE_IN_BOUNDS)


@functools.partial(
    pl.kernel,
    out_type=[
        jax.ShapeDtypeStruct((E,), jnp.float32),
        jax.ShapeDtypeStruct((E,), jnp.float32),
    ],
    mesh=plsc.VectorSubcoreMesh(core_axis_name="c", subcore_axis_name="s"),
    scratch_types=[
        pltpu.VMEM((C,), jnp.int32),
        pltpu.VMEM((C,), jnp.int32),
        pltpu.VMEM((C, D), jnp.float32),
        pltpu.VMEM((C, D), jnp.float32),
        pltpu.VMEM((C,), jnp.float32),
        pltpu.SemaphoreType.DMA,
        pltpu.SemaphoreType.DMA,
    ],
)
def _edge_scores(user_hbm, item_hbm, pu_hbm, pi_hbm, nu_hbm, ni_hbm,
                 pos_out, neg_out,
                 uidx_v, iidx_v, urows_v, vrows_v, out_v, sem0, sem1):
    wid = lax.axis_index("s") * NC + lax.axis_index("c")
    lane = lax.iota(jnp.int32, L)
    perms = [jnp.bitwise_xor(lane, s) for s in (8, 4, 2, 1)]
    nchunks_w = jnp.where(wid < EXTRA, BASE_PER_W + 1, BASE_PER_W)

    for u_hbm, i_hbm, o_hbm in ((pu_hbm, pi_hbm, pos_out),
                                (nu_hbm, ni_hbm, neg_out)):
        def chunk_body(k, _, u_hbm=u_hbm, i_hbm=i_hbm, o_hbm=o_hbm):
            base = (wid + k * NW) * C
            pltpu.sync_copy(u_hbm.at[pl.ds(base, C)], uidx_v)
            pltpu.sync_copy(i_hbm.at[pl.ds(base, C)], iidx_v)
            cp_u = pltpu.async_copy(user_hbm.at[uidx_v], urows_v, sem0)
            cp_i = pltpu.async_copy(item_hbm.at[iidx_v], vrows_v, sem1)
            cp_u.wait()
            cp_i.wait()

            def grp_body(g, _):
                out_vec = jnp.zeros((L,), jnp.float32)
                for e in range(L):
                    row = g * L + e
                    acc = urows_v[row, pl.ds(0, L)] * vrows_v[row, pl.ds(0, L)]
                    for kk in range(1, D // L):
                        a = urows_v[row, pl.ds(kk * L, L)]
                        b = vrows_v[row, pl.ds(kk * L, L)]
                        acc = acc + a * b
                    for p in perms:
                        acc = acc + _permute(acc, p)
                    out_vec = jnp.where(lane == e, acc, out_vec)
                out_v[pl.ds(g * L, L)] = out_vec
                return 0

            lax.fori_loop(0, C // L, grp_body, 0)
            pltpu.sync_copy(out_v, o_hbm.at[pl.ds(base, C)])
            return 0

        lax.fori_loop(0, nchunks_w, chunk_body, 0)


def kernel(user_embedding, item_embedding, pos_edges, neg_edges):
    pu, pi = pos_edges[0], pos_edges[1]
    nu, ni = neg_edges[0], neg_edges[1]
    score_pos, score_neg = _edge_scores(user_embedding, item_embedding,
                                        pu, pi, nu, ni)
    return (score_pos[:, None], score_neg[:, None])
